# Initial kernel scaffold; baseline (speedup 1.0000x reference)
import jax
import jax.numpy as jnp
from jax.experimental import pallas as pl


def kernel(batch, lengths, table):
    raise NotImplementedError("write your pallas kernel here")


# speedup vs baseline: 4.9419x; 4.9419x over previous
import functools

import jax
import jax.numpy as jnp
from jax import lax
from jax.experimental import pallas as pl
from jax.experimental.pallas import tpu as pltpu
from jax.experimental.pallas import tpu_sc as plsc

_INFO = plsc.get_sparse_core_info()
_NC = _INFO.num_cores
_NS = _INFO.num_subcores
_NW = _NC * _NS

_CHUNK = 2048


@functools.partial(jax.jit, static_argnums=(2, 3))
def _gather_rows(table, idx, n, d):
    per_w = n // _NW
    n_chunks = per_w // _CHUNK
    mesh = plsc.VectorSubcoreMesh(core_axis_name="c", subcore_axis_name="s")

    @functools.partial(
        pl.kernel,
        mesh=mesh,
        out_type=jax.ShapeDtypeStruct((n, d), jnp.float32),
        scratch_types=[
            pltpu.VMEM((_CHUNK,), jnp.int32),
            pltpu.VMEM((_CHUNK, d), jnp.float32),
            pltpu.SemaphoreType.DMA,
        ],
        compiler_params=pltpu.CompilerParams(use_tc_tiling_on_sc=False),
    )
    def k(table_hbm, idx_hbm, out_hbm, idx_v, rows_v, sem):
        wid = lax.axis_index("s") * _NC + lax.axis_index("c")
        base = wid * per_w

        def body(g, carry):
            off = base + g * _CHUNK
            pltpu.sync_copy(idx_hbm.at[pl.ds(off, _CHUNK)], idx_v)
            pltpu.async_copy(table_hbm.at[idx_v], rows_v, sem).wait()
            pltpu.sync_copy(rows_v, out_hbm.at[pl.ds(off, _CHUNK)])
            return carry

        lax.fori_loop(0, n_chunks, body, 0)

    return k(table, idx)


def kernel(batch, lengths, table):
    b, l = batch.shape
    v, d = table.shape
    n = b * l
    out = _gather_rows(table, batch.reshape(n), n, d)
    return out.reshape(b, l, d)


# speedup vs baseline: 5.0450x; 1.0209x over previous
import functools

import jax
import jax.numpy as jnp
from jax import lax
from jax.experimental import pallas as pl
from jax.experimental.pallas import tpu as pltpu
from jax.experimental.pallas import tpu_sc as plsc

_INFO = plsc.get_sparse_core_info()
_NC = _INFO.num_cores
_NS = _INFO.num_subcores
_NW = _NC * _NS

_CHUNK = 800
_NBUF = 4


@functools.partial(jax.jit, static_argnums=(2, 3))
def _gather_rows(table, idx, n, d):
    per_w = n // _NW
    n_chunks = per_w // _CHUNK
    n_groups = n_chunks // _NBUF
    mesh = plsc.VectorSubcoreMesh(core_axis_name="c", subcore_axis_name="s")

    @functools.partial(
        pl.kernel,
        mesh=mesh,
        out_type=jax.ShapeDtypeStruct((n, d), jnp.float32),
        scratch_types=[
            pltpu.VMEM((_NBUF, _CHUNK), jnp.int32),
            pltpu.VMEM((_NBUF, _CHUNK, d), jnp.float32),
            pltpu.SemaphoreType.DMA((_NBUF,)),
            pltpu.SemaphoreType.DMA((_NBUF,)),
            pltpu.SemaphoreType.DMA((_NBUF,)),
        ],
        compiler_params=pltpu.CompilerParams(use_tc_tiling_on_sc=False),
    )
    def k(table_hbm, idx_hbm, out_hbm, idx_v, rows_v, sem_i, sem_g, sem_o):
        wid = lax.axis_index("s") * _NC + lax.axis_index("c")
        base = wid * per_w

        def start_idx(c, b):
            off = base + c * _CHUNK
            pltpu.async_copy(idx_hbm.at[pl.ds(off, _CHUNK)], idx_v.at[b],
                             sem_i.at[b])

        def wait_idx(b):
            pltpu.make_async_copy(idx_hbm.at[pl.ds(0, _CHUNK)], idx_v.at[b],
                                  sem_i.at[b]).wait()

        def start_gather(b):
            pltpu.async_copy(table_hbm.at[idx_v.at[b]], rows_v.at[b],
                             sem_g.at[b])

        def wait_gather(b):
            pltpu.make_async_copy(table_hbm.at[idx_v.at[b]], rows_v.at[b],
                                  sem_g.at[b]).wait()

        def start_store(c, b):
            off = base + c * _CHUNK
            pltpu.async_copy(rows_v.at[b], out_hbm.at[pl.ds(off, _CHUNK)],
                             sem_o.at[b])

        def wait_store(b):
            pltpu.make_async_copy(rows_v.at[b], out_hbm.at[pl.ds(0, _CHUNK)],
                                  sem_o.at[b]).wait()

        start_idx(0, 0)
        start_idx(1, 1)
        wait_idx(0)
        start_gather(0)

        def body(j, carry):
            for b0 in range(_NBUF):
                i = j * _NBUF + b0
                b2 = (b0 + 2) % _NBUF
                b1 = (b0 + 1) % _NBUF

                @pl.when(i + 2 < n_chunks)
                def _():
                    @pl.when(i >= _NBUF - 2)
                    def _():
                        wait_store(b2)
                    start_idx(i + 2, b2)

                @pl.when(i + 1 < n_chunks)
                def _():
                    wait_idx(b1)
                    start_gather(b1)

                wait_gather(b0)
                start_store(i, b0)
            return carry

        lax.fori_loop(0, n_groups, body, 0)

        for b in range(_NBUF):
            wait_store(b)

    return k(table, idx)


def kernel(batch, lengths, table):
    b, l = batch.shape
    v, d = table.shape
    n = b * l
    out = _gather_rows(table, batch.reshape(n), n, d)
    return out.reshape(b, l, d)
